# baseline (speedup 1.0000x reference)
import functools

import jax
import jax.numpy as jnp
from jax import lax
from jax.experimental import pallas as pl
from jax.experimental.pallas import tpu as pltpu
from jax.experimental.pallas import tpu_sc as plsc

_N = 10000
_E = 320000
_D = 128

_NC = 2
_NS = 16
_NW = _NC * _NS
_GROUP = 128
_SPT = 80
_NGP = _NW * _SPT
_EPAD = _NGP * _GROUP
_OCTS = _SPT // 8
_RPT = 632
_NPAD = _NS * _RPT


def _sc_spmm(x, pk, w, zeros):
    mesh = plsc.VectorSubcoreMesh(core_axis_name="c", subcore_axis_name="s")

    @functools.partial(
        pl.kernel,
        mesh=mesh,
        out_type=jax.ShapeDtypeStruct((_NC, _NPAD, _D), jnp.float32),
        scratch_types=[
            pltpu.VMEM((16, _GROUP), jnp.int32),
            pltpu.VMEM((8, _GROUP), jnp.float32),
            pltpu.VMEM((_GROUP, _D), jnp.float32),
            pltpu.VMEM_SHARED((_NPAD, _D), jnp.float32),
            pltpu.SemaphoreType.DMA,
        ],
    )
    def k(x_hbm, pk_hbm, w_hbm, z_hbm, out_hbm,
          pk_v, w_v, rows_v, acc_sh, sem):
        cid = lax.axis_index("c")
        sid = lax.axis_index("s")
        wid = sid * _NC + cid

        pltpu.sync_copy(z_hbm, acc_sh.at[pl.ds(sid * _RPT, _RPT)])
        plsc.subcore_barrier()

        def oct_body(o, carry):
            pltpu.sync_copy(pk_hbm.at[wid, pl.ds(o * 16, 16)], pk_v)
            pltpu.sync_copy(w_hbm.at[wid, pl.ds(o * 8, 8)], w_v)
            for j in range(8):
                pltpu.async_copy(x_hbm.at[pk_v.at[2 * j]], rows_v,
                                 sem).wait()

                def escale(s, c2, j=j):
                    wv16 = w_v[j, pl.ds(s * 16, 16)]
                    for i in range(16):
                        e = s * 16 + i
                        wv = jnp.full((16,), wv16[i], dtype=jnp.float32)
                        for dd in range(_D // 16):
                            sl = pl.ds(dd * 16, 16)
                            rows_v[e, sl] = rows_v[e, sl] * wv
                    return c2

                lax.fori_loop(0, _GROUP // 16, escale, 0)
                pltpu.sync_copy(rows_v, acc_sh.at[pk_v.at[2 * j + 1]],
                                add=True)
            return carry

        lax.fori_loop(0, _OCTS, oct_body, 0)
        plsc.subcore_barrier()
        pltpu.sync_copy(acc_sh.at[pl.ds(sid * _RPT, _RPT)],
                        out_hbm.at[cid, pl.ds(sid * _RPT, _RPT)])

    return k(x, pk, w, zeros)


def _add_body(a_ref, o_ref):
    o_ref[...] = a_ref[0] + a_ref[1]


def _combine(partials):
    blk = 1000
    return pl.pallas_call(
        _add_body,
        grid=(_N // blk,),
        in_specs=[pl.BlockSpec((_NC, blk, _D), lambda i: (0, i, 0))],
        out_specs=pl.BlockSpec((blk, _D), lambda i: (i, 0)),
        out_shape=jax.ShapeDtypeStruct((_N, _D), jnp.float32),
    )(partials)


def kernel(x, edge_index, edge_weight):
    pad = _EPAD - _E
    col = jnp.concatenate([edge_index[1], jnp.zeros((pad,), jnp.int32)])
    row = jnp.concatenate([edge_index[0], jnp.zeros((pad,), jnp.int32)])
    wp = jnp.concatenate([edge_weight, jnp.zeros((pad,), jnp.float32)])
    colr = col.reshape(_SPT, _NW, _GROUP)
    rowr = row.reshape(_SPT, _NW, _GROUP)
    pk = jnp.stack([colr, rowr], axis=2).transpose(1, 0, 2, 3).reshape(
        _NW, 2 * _SPT, _GROUP)
    w2 = wp.reshape(_SPT, _NW, _GROUP).transpose(1, 0, 2)
    zeros = jnp.zeros((_RPT, _D), jnp.float32)
    partials = _sc_spmm(x, pk, w2, zeros)
    return _combine(partials)


# speedup vs baseline: 1.0019x; 1.0019x over previous
import functools

import jax
import jax.numpy as jnp
from jax import lax
from jax.experimental import pallas as pl
from jax.experimental.pallas import tpu as pltpu
from jax.experimental.pallas import tpu_sc as plsc

_N = 10000
_E = 320000
_D = 128

_NC = 2
_NS = 16
_NW = _NC * _NS
_GROUP = 128
_SPT = 80
_NGP = _NW * _SPT
_EPAD = _NGP * _GROUP
_OCTS = _SPT // 8
_RPT = 632
_NPAD = _NS * _RPT


def _sc_spmm(x, pk, w, zeros):
    mesh = plsc.VectorSubcoreMesh(core_axis_name="c", subcore_axis_name="s")

    @functools.partial(
        pl.kernel,
        mesh=mesh,
        out_type=jax.ShapeDtypeStruct((_NC, _NPAD, _D), jnp.float32),
        scratch_types=[
            pltpu.VMEM((16, _GROUP), jnp.int32),
            pltpu.VMEM((8, _GROUP), jnp.float32),
            pltpu.VMEM((_GROUP, _D), jnp.float32),
            pltpu.VMEM_SHARED((_NPAD, _D), jnp.float32),
            pltpu.SemaphoreType.DMA,
        ],
    )
    def k(x_hbm, pk_hbm, w_hbm, z_hbm, out_hbm,
          pk_v, w_v, rows_v, acc_sh, sem):
        cid = lax.axis_index("c")
        sid = lax.axis_index("s")
        wid = sid * _NC + cid

        pltpu.sync_copy(z_hbm, acc_sh.at[pl.ds(sid * _RPT, _RPT)])
        plsc.subcore_barrier()

        def oct_body(o, carry):
            pltpu.sync_copy(pk_hbm.at[wid, pl.ds(o * 16, 16)], pk_v)
            pltpu.sync_copy(w_hbm.at[wid, pl.ds(o * 8, 8)], w_v)

            def group_body(j, c1):
                pltpu.async_copy(x_hbm.at[pk_v.at[2 * j]], rows_v,
                                 sem).wait()

                def escale(s, c2):
                    wv16 = w_v[j, pl.ds(s * 16, 16)]
                    for i in range(16):
                        e = s * 16 + i
                        wv = jnp.full((16,), wv16[i], dtype=jnp.float32)
                        for dd in range(_D // 16):
                            sl = pl.ds(dd * 16, 16)
                            rows_v[e, sl] = rows_v[e, sl] * wv
                    return c2

                lax.fori_loop(0, _GROUP // 16, escale, 0)
                pltpu.sync_copy(rows_v, acc_sh.at[pk_v.at[2 * j + 1]],
                                add=True)
                return c1

            lax.fori_loop(0, 8, group_body, 0)
            return carry

        lax.fori_loop(0, _OCTS, oct_body, 0)
        plsc.subcore_barrier()
        pltpu.sync_copy(acc_sh.at[pl.ds(sid * _RPT, _RPT)],
                        out_hbm.at[cid, pl.ds(sid * _RPT, _RPT)])

    return k(x, pk, w, zeros)


def _add_body(a_ref, o_ref):
    o_ref[...] = a_ref[0] + a_ref[1]


def _combine(partials):
    blk = 1000
    return pl.pallas_call(
        _add_body,
        grid=(_N // blk,),
        in_specs=[pl.BlockSpec((_NC, blk, _D), lambda i: (0, i, 0))],
        out_specs=pl.BlockSpec((blk, _D), lambda i: (i, 0)),
        out_shape=jax.ShapeDtypeStruct((_N, _D), jnp.float32),
    )(partials)


def kernel(x, edge_index, edge_weight):
    pad = _EPAD - _E
    col = jnp.concatenate([edge_index[1], jnp.zeros((pad,), jnp.int32)])
    row = jnp.concatenate([edge_index[0], jnp.zeros((pad,), jnp.int32)])
    wp = jnp.concatenate([edge_weight, jnp.zeros((pad,), jnp.float32)])
    colr = col.reshape(_SPT, _NW, _GROUP)
    rowr = row.reshape(_SPT, _NW, _GROUP)
    pk = jnp.stack([colr, rowr], axis=2).transpose(1, 0, 2, 3).reshape(
        _NW, 2 * _SPT, _GROUP)
    w2 = wp.reshape(_SPT, _NW, _GROUP).transpose(1, 0, 2)
    zeros = jnp.zeros((_RPT, _D), jnp.float32)
    partials = _sc_spmm(x, pk, w2, zeros)
    return _combine(partials)


# speedup vs baseline: 1.6929x; 1.6897x over previous
import functools

import jax
import jax.numpy as jnp
from jax import lax
from jax.experimental import pallas as pl
from jax.experimental.pallas import tpu as pltpu
from jax.experimental.pallas import tpu_sc as plsc

_N = 10000
_E = 320000
_D = 128

_NC = 2
_NS = 16
_NW = _NC * _NS
_GROUP = 128
_NGROUPS = _E // _GROUP
_RPT = 632
_NPAD = _NS * _RPT


def _sc_spmm(x, packed, w, zeros):
    mesh = plsc.VectorSubcoreMesh(core_axis_name="c", subcore_axis_name="s")

    @functools.partial(
        pl.kernel,
        mesh=mesh,
        out_type=jax.ShapeDtypeStruct((_NC, _NPAD, _D), jnp.float32),
        scratch_types=[
            pltpu.VMEM((2, _GROUP), jnp.int32),
            pltpu.VMEM((_GROUP,), jnp.float32),
            pltpu.VMEM((_GROUP, _D), jnp.float32),
            pltpu.VMEM_SHARED((_NPAD, _D), jnp.float32),
            pltpu.SemaphoreType.DMA,
        ],
    )
    def k(x_hbm, pk_hbm, w_hbm, z_hbm, out_hbm,
          pk_v, w_v, rows_v, acc_sh, sem):
        cid = lax.axis_index("c")
        sid = lax.axis_index("s")
        wid = sid * _NC + cid

        pltpu.sync_copy(z_hbm, acc_sh.at[pl.ds(sid * _RPT, _RPT)])
        plsc.subcore_barrier()

        n_mine = (_NGROUPS - wid + _NW - 1) // _NW

        def group_body(t, carry):
            g = wid + t * _NW
            pltpu.sync_copy(pk_hbm.at[g], pk_v)
            pltpu.sync_copy(w_hbm.at[pl.ds(g * _GROUP, _GROUP)], w_v)
            pltpu.async_copy(x_hbm.at[pk_v.at[0]], rows_v, sem).wait()

            def escale(s, c2):
                wv16 = w_v[pl.ds(s * 16, 16)]
                for j in range(16):
                    e = s * 16 + j
                    wv = jnp.full((16,), wv16[j], dtype=jnp.float32)
                    for dd in range(_D // 16):
                        sl = pl.ds(dd * 16, 16)
                        rows_v[e, sl] = rows_v[e, sl] * wv
                return c2

            lax.fori_loop(0, _GROUP // 16, escale, 0)
            pltpu.sync_copy(rows_v, acc_sh.at[pk_v.at[1]], add=True)
            return carry

        lax.fori_loop(0, n_mine, group_body, 0)
        plsc.subcore_barrier()
        pltpu.sync_copy(acc_sh.at[pl.ds(sid * _RPT, _RPT)],
                        out_hbm.at[cid, pl.ds(sid * _RPT, _RPT)])

    return k(x, packed, w, zeros)


def _add_body(a_ref, o_ref):
    o_ref[...] = a_ref[0] + a_ref[1]


def _combine(partials):
    blk = 1000
    return pl.pallas_call(
        _add_body,
        grid=(_N // blk,),
        in_specs=[pl.BlockSpec((_NC, blk, _D), lambda i: (0, i, 0))],
        out_specs=pl.BlockSpec((blk, _D), lambda i: (i, 0)),
        out_shape=jax.ShapeDtypeStruct((_N, _D), jnp.float32),
    )(partials)


def kernel(x, edge_index, edge_weight):
    packed = jnp.stack([
        edge_index[1].reshape(_NGROUPS, _GROUP),
        edge_index[0].reshape(_NGROUPS, _GROUP)],
        axis=1)
    zeros = jnp.zeros((_RPT, _D), jnp.float32)
    partials = _sc_spmm(x, packed, edge_weight, zeros)
    return _combine(partials)
